# baseline (speedup 1.0000x reference)
import jax
import jax.numpy as jnp
from jax import lax
from jax.experimental import pallas as pl
from jax.experimental.pallas import tpu as pltpu
from jax.experimental.pallas import tpu_sc as plsc

B = 16384
CLASS_NUM = 2
D = 1024
LANES = 16
NC = 2
NS = 16
NW = NC * NS
R = B // NW
CHUNK = 32
NCHUNK = R // CHUNK
CW = CHUNK * D
NJ = D // LANES


def _sc_body(feat_hbm, cen_hbm, out_hbm, cen_v, cd_v, fb0, fb1,
             a0p_v, pdp_v, sem0, sem1):
    wid = lax.axis_index("s") * NC + lax.axis_index("c")
    base = wid * R

    pltpu.sync_copy(cen_hbm, cen_v)
    for j in range(NJ):
        c0 = cen_v[pl.ds(j * LANES, LANES)]
        c1 = cen_v[pl.ds(D + j * LANES, LANES)]
        cd_v[pl.ds(j * LANES, LANES)] = c0 - c1

    def process(g, fb):
        def row_body(r, _):
            ridx = g * CHUNK + r
            a0 = jnp.zeros((LANES,), jnp.float32)
            pd = jnp.zeros((LANES,), jnp.float32)
            for j in range(NJ):
                f = fb[pl.ds(r * D + j * LANES, LANES)]
                c0 = cen_v[pl.ds(j * LANES, LANES)]
                cd = cd_v[pl.ds(j * LANES, LANES)]
                diff = f - c0
                a0 = a0 + diff * diff
                pd = pd + f * cd
            a0p_v[pl.ds(ridx * LANES, LANES)] = a0
            pdp_v[pl.ds(ridx * LANES, LANES)] = pd
            return 0

        lax.fori_loop(0, CHUNK, row_body, 0)

    pltpu.async_copy(feat_hbm.at[pl.ds(base * D, CW)], fb0, sem0)

    def pair_body(k, _):
        g0 = 2 * k
        g1 = 2 * k + 1
        pltpu.async_copy(
            feat_hbm.at[pl.ds((base + g1 * CHUNK) * D, CW)], fb1, sem1)
        pltpu.make_async_copy(feat_hbm.at[pl.ds(0, CW)], fb0, sem0).wait()
        process(g0, fb0)
        nxt = jnp.minimum(g0 + 2, NCHUNK - 1)
        pltpu.async_copy(
            feat_hbm.at[pl.ds((base + nxt * CHUNK) * D, CW)], fb0, sem0)
        pltpu.make_async_copy(feat_hbm.at[pl.ds(0, CW)], fb1, sem1).wait()
        process(g1, fb1)
        return 0

    lax.fori_loop(0, NCHUNK // 2, pair_body, 0)
    pltpu.make_async_copy(feat_hbm.at[pl.ds(0, CW)], fb0, sem0).wait()

    pltpu.sync_copy(a0p_v, out_hbm.at[pl.ds(base * LANES, R * LANES)])
    pltpu.sync_copy(pdp_v,
                    out_hbm.at[pl.ds(B * LANES + base * LANES, R * LANES)])


def _sc_partials(feat_flat, cen_flat):
    mesh = plsc.VectorSubcoreMesh(core_axis_name="c", subcore_axis_name="s")
    return pl.kernel(
        _sc_body,
        mesh=mesh,
        out_type=jax.ShapeDtypeStruct((2 * B * LANES,), jnp.float32),
        scratch_types=[
            pltpu.VMEM((CLASS_NUM * D,), jnp.float32),
            pltpu.VMEM((D,), jnp.float32),
            pltpu.VMEM((CW,), jnp.float32),
            pltpu.VMEM((CW,), jnp.float32),
            pltpu.VMEM((R * LANES,), jnp.float32),
            pltpu.VMEM((R * LANES,), jnp.float32),
            pltpu.SemaphoreType.DMA,
            pltpu.SemaphoreType.DMA,
        ],
    )(feat_flat, cen_flat)


TBLK = 2048
TNBLK = B // TBLK


def _tc_finish_body(tag_ref, a0p_ref, pdp_ref, cen_ref, out_ref, acc_ref):
    i = pl.program_id(0)
    t = tag_ref[0, 0, :].astype(jnp.float32)
    a0 = jnp.sum(a0p_ref[...], axis=1)
    pd = jnp.sum(pdp_ref[...], axis=1)
    cs = jnp.sum(cen_ref[...] * cen_ref[...], axis=1)
    dcs = cs[1] - cs[0]
    d2 = a0 + t * (2.0 * pd + dcs)
    d = jnp.sqrt(jnp.maximum(d2, 0.0))
    s1 = jnp.sum(d * t)
    s_all = jnp.sum(d)
    n1 = jnp.sum(t)

    @pl.when(i == 0)
    def _():
        acc_ref[0] = 0.0
        acc_ref[1] = 0.0
        acc_ref[2] = 0.0

    acc_ref[0] += s_all - s1
    acc_ref[1] += s1
    acc_ref[2] += n1

    @pl.when(i == TNBLK - 1)
    def _():
        s0_t = acc_ref[0]
        s1_t = acc_ref[1]
        n1_t = acc_ref[2]
        n0_t = jnp.float32(B) - n1_t
        l0 = jnp.where(n0_t > 0, s0_t / jnp.maximum(n0_t, 1.0), 0.0)
        l1 = jnp.where(n1_t > 0, s1_t / jnp.maximum(n1_t, 1.0), 0.0)
        out_ref[0] = l0 + l1


def kernel(tag, feature, center):
    parts = _sc_partials(feature.reshape(-1), center.reshape(-1))
    a0p = parts[:B * LANES].reshape(B, LANES)
    pdp = parts[B * LANES:].reshape(B, LANES)
    out = pl.pallas_call(
        _tc_finish_body,
        grid=(TNBLK,),
        in_specs=[
            pl.BlockSpec((1, 1, TBLK), lambda i: (i, 0, 0)),
            pl.BlockSpec((TBLK, LANES), lambda i: (i, 0)),
            pl.BlockSpec((TBLK, LANES), lambda i: (i, 0)),
            pl.BlockSpec((CLASS_NUM, D), lambda i: (0, 0)),
        ],
        out_specs=pl.BlockSpec(memory_space=pltpu.MemorySpace.SMEM),
        out_shape=jax.ShapeDtypeStruct((1,), jnp.float32),
        scratch_shapes=[pltpu.SMEM((3,), jnp.float32)],
    )(tag.reshape(TNBLK, 1, TBLK), a0p, pdp, center)
    return out[0]


# speedup vs baseline: 1.3696x; 1.3696x over previous
import jax
import jax.numpy as jnp
from jax import lax
from jax.experimental import pallas as pl
from jax.experimental.pallas import tpu as pltpu
from jax.experimental.pallas import tpu_sc as plsc

B = 16384
CLASS_NUM = 2
D = 1024
LANES = 16
NC = 2
NS = 16
NW = NC * NS
R = B // NW
CHUNK = 32
NCHUNK = R // CHUNK
CW = CHUNK * D
NJ = D // LANES


def _sc_body(feat_hbm, cen_hbm, out_hbm, cen_v, cd_v, fb0, fb1,
             a0p_v, pdp_v, sem0, sem1):
    wid = lax.axis_index("s") * NC + lax.axis_index("c")
    base = wid * R

    pltpu.sync_copy(cen_hbm, cen_v)
    for j in range(NJ):
        c0 = cen_v[pl.ds(j * LANES, LANES)]
        c1 = cen_v[pl.ds(D + j * LANES, LANES)]
        cd_v[pl.ds(j * LANES, LANES)] = c0 - c1

    def process(g, fb):
        def row_body(r, _):
            ridx = g * CHUNK + r
            a0 = jnp.zeros((LANES,), jnp.float32)
            pd = jnp.zeros((LANES,), jnp.float32)
            for j in range(NJ):
                f = fb[r, pl.ds(j * LANES, LANES)]
                c0 = cen_v[pl.ds(j * LANES, LANES)]
                cd = cd_v[pl.ds(j * LANES, LANES)]
                diff = f - c0
                a0 = a0 + diff * diff
                pd = pd + f * cd
            a0p_v[pl.ds(ridx * LANES, LANES)] = a0
            pdp_v[pl.ds(ridx * LANES, LANES)] = pd
            return 0

        lax.fori_loop(0, CHUNK, row_body, 0)

    pltpu.async_copy(feat_hbm.at[pl.ds(base, CHUNK), :], fb0, sem0)

    def pair_body(k, _):
        g0 = 2 * k
        g1 = 2 * k + 1
        pltpu.async_copy(
            feat_hbm.at[pl.ds(base + g1 * CHUNK, CHUNK), :], fb1, sem1)
        pltpu.make_async_copy(
            feat_hbm.at[pl.ds(0, CHUNK), :], fb0, sem0).wait()
        process(g0, fb0)
        nxt = jnp.minimum(g0 + 2, NCHUNK - 1)
        pltpu.async_copy(
            feat_hbm.at[pl.ds(base + nxt * CHUNK, CHUNK), :], fb0, sem0)
        pltpu.make_async_copy(
            feat_hbm.at[pl.ds(0, CHUNK), :], fb1, sem1).wait()
        process(g1, fb1)
        return 0

    lax.fori_loop(0, NCHUNK // 2, pair_body, 0)
    pltpu.make_async_copy(feat_hbm.at[pl.ds(0, CHUNK), :], fb0, sem0).wait()

    pltpu.sync_copy(a0p_v, out_hbm.at[pl.ds(base * LANES, R * LANES)])
    pltpu.sync_copy(pdp_v,
                    out_hbm.at[pl.ds(B * LANES + base * LANES, R * LANES)])


def _sc_partials(feat, cen_flat):
    mesh = plsc.VectorSubcoreMesh(core_axis_name="c", subcore_axis_name="s")
    return pl.kernel(
        _sc_body,
        mesh=mesh,
        out_type=jax.ShapeDtypeStruct((2 * B * LANES,), jnp.float32),
        scratch_types=[
            pltpu.VMEM((CLASS_NUM * D,), jnp.float32),
            pltpu.VMEM((D,), jnp.float32),
            pltpu.VMEM((CHUNK, D), jnp.float32),
            pltpu.VMEM((CHUNK, D), jnp.float32),
            pltpu.VMEM((R * LANES,), jnp.float32),
            pltpu.VMEM((R * LANES,), jnp.float32),
            pltpu.SemaphoreType.DMA,
            pltpu.SemaphoreType.DMA,
        ],
    )(feat, cen_flat)


TBLK = 2048
TNBLK = B // TBLK


def _tc_finish_body(tag_ref, a0p_ref, pdp_ref, cen_ref, out_ref, acc_ref):
    i = pl.program_id(0)
    t = tag_ref[0, 0, :].astype(jnp.float32)
    a0 = jnp.sum(a0p_ref[...], axis=1)
    pd = jnp.sum(pdp_ref[...], axis=1)
    cs = jnp.sum(cen_ref[...] * cen_ref[...], axis=1)
    dcs = cs[1] - cs[0]
    d2 = a0 + t * (2.0 * pd + dcs)
    d = jnp.sqrt(jnp.maximum(d2, 0.0))
    s1 = jnp.sum(d * t)
    s_all = jnp.sum(d)
    n1 = jnp.sum(t)

    @pl.when(i == 0)
    def _():
        acc_ref[0] = 0.0
        acc_ref[1] = 0.0
        acc_ref[2] = 0.0

    acc_ref[0] += s_all - s1
    acc_ref[1] += s1
    acc_ref[2] += n1

    @pl.when(i == TNBLK - 1)
    def _():
        s0_t = acc_ref[0]
        s1_t = acc_ref[1]
        n1_t = acc_ref[2]
        n0_t = jnp.float32(B) - n1_t
        l0 = jnp.where(n0_t > 0, s0_t / jnp.maximum(n0_t, 1.0), 0.0)
        l1 = jnp.where(n1_t > 0, s1_t / jnp.maximum(n1_t, 1.0), 0.0)
        out_ref[0] = l0 + l1


def kernel(tag, feature, center):
    parts = _sc_partials(feature, center.reshape(-1))
    a0p = parts[:B * LANES].reshape(B, LANES)
    pdp = parts[B * LANES:].reshape(B, LANES)
    out = pl.pallas_call(
        _tc_finish_body,
        grid=(TNBLK,),
        in_specs=[
            pl.BlockSpec((1, 1, TBLK), lambda i: (i, 0, 0)),
            pl.BlockSpec((TBLK, LANES), lambda i: (i, 0)),
            pl.BlockSpec((TBLK, LANES), lambda i: (i, 0)),
            pl.BlockSpec((CLASS_NUM, D), lambda i: (0, 0)),
        ],
        out_specs=pl.BlockSpec(memory_space=pltpu.MemorySpace.SMEM),
        out_shape=jax.ShapeDtypeStruct((1,), jnp.float32),
        scratch_shapes=[pltpu.SMEM((3,), jnp.float32)],
    )(tag.reshape(TNBLK, 1, TBLK), a0p, pdp, center)
    return out[0]


# speedup vs baseline: 7.8055x; 5.6990x over previous
import jax
import jax.numpy as jnp
from jax.experimental import pallas as pl
from jax.experimental.pallas import tpu as pltpu

B = 16384
CLASS_NUM = 2
FEATURE_DIM = 1024
BLK = 2048
NBLK = B // BLK


def _body(tag_ref, feat_ref, center_ref, out_ref, acc_ref):
    i = pl.program_id(0)
    t = tag_ref[0, 0, :]
    f = feat_ref[...]
    c0 = center_ref[0, :]
    c1 = center_ref[1, :]
    sel = (t[:, None] == 0)
    c = jnp.where(sel, c0[None, :], c1[None, :])
    diff = f - c
    q = jnp.sum(diff * diff, axis=1)
    d = jnp.sqrt(q)
    tf = t.astype(jnp.float32)
    s1 = jnp.sum(d * tf)
    s_all = jnp.sum(d)
    n1 = jnp.sum(tf)

    @pl.when(i == 0)
    def _():
        acc_ref[0] = 0.0
        acc_ref[1] = 0.0
        acc_ref[2] = 0.0

    acc_ref[0] += s_all - s1
    acc_ref[1] += s1
    acc_ref[2] += n1

    @pl.when(i == NBLK - 1)
    def _():
        s0_t = acc_ref[0]
        s1_t = acc_ref[1]
        n1_t = acc_ref[2]
        n0_t = jnp.float32(B) - n1_t
        l0 = jnp.where(n0_t > 0, s0_t / jnp.maximum(n0_t, 1.0), 0.0)
        l1 = jnp.where(n1_t > 0, s1_t / jnp.maximum(n1_t, 1.0), 0.0)
        out_ref[0] = l0 + l1


def kernel(tag, feature, center):
    tag3 = tag.reshape(NBLK, 1, BLK)
    out = pl.pallas_call(
        _body,
        grid=(NBLK,),
        in_specs=[
            pl.BlockSpec((1, 1, BLK), lambda i: (i, 0, 0)),
            pl.BlockSpec((BLK, FEATURE_DIM), lambda i: (i, 0)),
            pl.BlockSpec((CLASS_NUM, FEATURE_DIM), lambda i: (0, 0)),
        ],
        out_specs=pl.BlockSpec(memory_space=pltpu.MemorySpace.SMEM),
        out_shape=jax.ShapeDtypeStruct((1,), jnp.float32),
        scratch_shapes=[pltpu.SMEM((3,), jnp.float32)],
    )(tag3, feature, center)
    return out[0]


# speedup vs baseline: 7.9474x; 1.0182x over previous
import jax
import jax.numpy as jnp
from jax.experimental import pallas as pl
from jax.experimental.pallas import tpu as pltpu

B = 16384
CLASS_NUM = 2
FEATURE_DIM = 1024
BLK = 4096
NBLK = B // BLK


def _body(tag_ref, feat_ref, center_ref, out_ref, acc_ref):
    i = pl.program_id(0)
    t = tag_ref[0, 0, :]
    f = feat_ref[...]
    c0 = center_ref[0, :]
    c1 = center_ref[1, :]
    sel = (t[:, None] == 0)
    c = jnp.where(sel, c0[None, :], c1[None, :])
    diff = f - c
    q = jnp.sum(diff * diff, axis=1)
    d = jnp.sqrt(q)
    tf = t.astype(jnp.float32)
    s1 = jnp.sum(d * tf)
    s_all = jnp.sum(d)
    n1 = jnp.sum(tf)

    @pl.when(i == 0)
    def _():
        acc_ref[0] = 0.0
        acc_ref[1] = 0.0
        acc_ref[2] = 0.0

    acc_ref[0] += s_all - s1
    acc_ref[1] += s1
    acc_ref[2] += n1

    @pl.when(i == NBLK - 1)
    def _():
        s0_t = acc_ref[0]
        s1_t = acc_ref[1]
        n1_t = acc_ref[2]
        n0_t = jnp.float32(B) - n1_t
        l0 = jnp.where(n0_t > 0, s0_t / jnp.maximum(n0_t, 1.0), 0.0)
        l1 = jnp.where(n1_t > 0, s1_t / jnp.maximum(n1_t, 1.0), 0.0)
        out_ref[0] = l0 + l1


def kernel(tag, feature, center):
    tag3 = tag.reshape(NBLK, 1, BLK)
    out = pl.pallas_call(
        _body,
        grid=(NBLK,),
        in_specs=[
            pl.BlockSpec((1, 1, BLK), lambda i: (i, 0, 0)),
            pl.BlockSpec((BLK, FEATURE_DIM), lambda i: (i, 0)),
            pl.BlockSpec((CLASS_NUM, FEATURE_DIM), lambda i: (0, 0)),
        ],
        out_specs=pl.BlockSpec(memory_space=pltpu.MemorySpace.SMEM),
        out_shape=jax.ShapeDtypeStruct((1,), jnp.float32),
        scratch_shapes=[pltpu.SMEM((3,), jnp.float32)],
    )(tag3, feature, center)
    return out[0]
